# baseline (speedup 1.0000x reference)
import functools

import jax
import jax.numpy as jnp
from jax import lax
from jax.experimental import pallas as pl
from jax.experimental.pallas import tpu as pltpu
from jax.experimental.pallas import tpu_sc as plsc

_N = 10000
_NPAD = 10240
_K = 128
_NW = 32
_NSUB = 16
_BLK = 1000


def _sc_degree(dstp):
    ch = dstp.shape[1]
    rpt = _NPAD // _NSUB
    mesh = plsc.VectorSubcoreMesh(core_axis_name="c", subcore_axis_name="s")

    @functools.partial(
        pl.kernel,
        out_type=jax.ShapeDtypeStruct((2, _NPAD, 16), jnp.float32),
        mesh=mesh,
        scratch_types=[
            pltpu.VMEM((ch, _K), jnp.int32),
            pltpu.VMEM((_K, 16), jnp.float32),
            pltpu.VMEM((16, 16), jnp.float32),
            pltpu.VMEM_SHARED((_NPAD, 16), jnp.float32),
        ],
    )
    def k(dst_h, out_h, dst_v, ones_v, zero_v, acc):
        cid = lax.axis_index("c")
        sid = lax.axis_index("s")
        wid = cid * _NSUB + sid

        one = jnp.ones((16,), jnp.float32)
        zero = jnp.zeros((16,), jnp.float32)
        for r in range(_K):
            ones_v[r, pl.ds(0, 16)] = one
        for r in range(16):
            zero_v[r, pl.ds(0, 16)] = zero

        pltpu.sync_copy(dst_h.at[wid], dst_v)

        def zbody(z, c):
            pltpu.sync_copy(zero_v, acc.at[pl.ds(sid * rpt + z * 16, 16)])
            return c
        lax.fori_loop(0, rpt // 16, zbody, 0)
        plsc.subcore_barrier()

        def ebody(j, c):
            pltpu.sync_copy(ones_v, acc.at[dst_v.at[j]], add=True)
            return c
        lax.fori_loop(0, ch, ebody, 0)
        plsc.subcore_barrier()

        pltpu.sync_copy(acc.at[pl.ds(sid * rpt, rpt)],
                        out_h.at[cid, pl.ds(sid * rpt, rpt)])

    return k(dstp)


def _sc_scatter(table, sdp, d):
    ch = sdp.shape[0] // _NW
    rpt = _NPAD // _NSUB
    zr = 16
    mesh = plsc.VectorSubcoreMesh(core_axis_name="c", subcore_axis_name="s")

    @functools.partial(
        pl.kernel,
        out_type=jax.ShapeDtypeStruct((2, _NPAD, d), jnp.float32),
        mesh=mesh,
        scratch_types=[
            pltpu.VMEM((2, _K), jnp.int32),
            pltpu.VMEM((2, _K), jnp.int32),
            pltpu.VMEM((2, _K), jnp.int32),
            pltpu.VMEM((2, _K), jnp.int32),
            pltpu.VMEM((_K, d), jnp.float32),
            pltpu.VMEM((_K, d), jnp.float32),
            pltpu.VMEM((zr, d), jnp.float32),
            pltpu.VMEM_SHARED((_NPAD, d), jnp.float32),
            pltpu.SemaphoreType.DMA,
            pltpu.SemaphoreType.DMA,
            pltpu.SemaphoreType.DMA,
            pltpu.SemaphoreType.DMA,
            pltpu.SemaphoreType.DMA,
            pltpu.SemaphoreType.DMA,
        ],
        compiler_params=pltpu.CompilerParams(use_tc_tiling_on_sc=False),
    )
    def k(table_h, sd_h, out_h, i0, i1, i2, i3, r0, r1, zero_v, acc,
          is0, is1, is2, is3, gs0, gs1):
        cid = lax.axis_index("c")
        sid = lax.axis_index("s")
        wid = cid * _NSUB + sid
        slot = (i0, i1, i2, i3)
        isem = (is0, is1, is2, is3)
        rows = (r0, r1)
        gsem = (gs0, gs1)

        zero = jnp.zeros((16,), jnp.float32)
        for r in range(zr):
            for cc in range(d // 16):
                zero_v[r, pl.ds(cc * 16, 16)] = zero

        for i in range(4):
            pltpu.async_copy(sd_h.at[wid * ch + i], slot[i], isem[i])

        def zbody(z, c):
            pltpu.sync_copy(zero_v, acc.at[pl.ds(sid * rpt + z * zr, zr)])
            return c
        lax.fori_loop(0, rpt // zr, zbody, 0)
        plsc.subcore_barrier()

        for b in range(2):
            pltpu.make_async_copy(sd_h.at[wid * ch + b], slot[b], isem[b]).wait()
            pltpu.async_copy(table_h.at[slot[b].at[0]], rows[b], gsem[b])

        def ebody(g, c):
            for u in range(4):
                j = g * 4 + u
                b = u % 2
                pltpu.make_async_copy(
                    table_h.at[slot[u].at[0]], rows[b], gsem[b]).wait()
                pltpu.sync_copy(rows[b], acc.at[slot[u].at[1]], add=True)

                @pl.when(j + 4 < ch)
                def _():
                    pltpu.async_copy(sd_h.at[wid * ch + j + 4], slot[u], isem[u])

                @pl.when(j + 2 < ch)
                def _():
                    un = (u + 2) % 4
                    pltpu.make_async_copy(
                        sd_h.at[wid * ch + j + 2], slot[un], isem[un]).wait()
                    pltpu.async_copy(
                        table_h.at[slot[un].at[0]], rows[b], gsem[b])
            return c
        lax.fori_loop(0, ch // 4, ebody, 0)
        plsc.subcore_barrier()

        pltpu.sync_copy(acc.at[pl.ds(sid * rpt, rpt)],
                        out_h.at[cid, pl.ds(sid * rpt, rpt)])

    return k(table, sdp)


def _tc1(x, w1, degn):
    def body(x_ref, w_ref, dg_ref, h_ref, dv_ref):
        deg = dg_ref[0] + dg_ref[1] + 1.0
        dinv = lax.rsqrt(deg)
        h_ref[...] = jnp.dot(x_ref[...], w_ref[...],
                             preferred_element_type=jnp.float32) * dinv
        dv_ref[...] = dinv

    return pl.pallas_call(
        body,
        grid=(_N // _BLK,),
        in_specs=[
            pl.BlockSpec((_BLK, 128), lambda i: (i, 0)),
            pl.BlockSpec((128, 128), lambda i: (0, 0)),
            pl.BlockSpec((2, _BLK, 1), lambda i: (0, i, 0)),
        ],
        out_specs=[
            pl.BlockSpec((_BLK, 128), lambda i: (i, 0)),
            pl.BlockSpec((_BLK, 1), lambda i: (i, 0)),
        ],
        out_shape=[
            jax.ShapeDtypeStruct((_N, 128), jnp.float32),
            jax.ShapeDtypeStruct((_N, 1), jnp.float32),
        ],
    )(x, w1, degn)


def _tc2(p1, h1p, dinv, b1, w2):
    def body(p_ref, h_ref, dv_ref, b_ref, w_ref, o_ref):
        dinv = dv_ref[...]
        z = dinv * (p_ref[0] + p_ref[1] + h_ref[...]) + b_ref[...]
        h2 = jnp.maximum(z, 0.0)
        o_ref[...] = jnp.dot(h2, w_ref[...],
                             preferred_element_type=jnp.float32) * dinv

    return pl.pallas_call(
        body,
        grid=(_N // _BLK,),
        in_specs=[
            pl.BlockSpec((2, _BLK, 128), lambda i: (0, i, 0)),
            pl.BlockSpec((_BLK, 128), lambda i: (i, 0)),
            pl.BlockSpec((_BLK, 1), lambda i: (i, 0)),
            pl.BlockSpec((1, 128), lambda i: (0, 0)),
            pl.BlockSpec((128, 64), lambda i: (0, 0)),
        ],
        out_specs=pl.BlockSpec((_BLK, 64), lambda i: (i, 0)),
        out_shape=jax.ShapeDtypeStruct((_N, 64), jnp.float32),
    )(p1, h1p, dinv, b1, w2)


def _tc3(p2, g2p, dinv, b2):
    def body(p_ref, g_ref, dv_ref, b_ref, o_ref):
        z = dv_ref[...] * (p_ref[0] + p_ref[1] + g_ref[...]) + b_ref[...]
        m = jnp.max(z, axis=1, keepdims=True)
        e = jnp.exp(z - m)
        s = jnp.sum(e, axis=1, keepdims=True)
        o_ref[...] = (z - m) - jnp.log(s)

    return pl.pallas_call(
        body,
        grid=(_N // _BLK,),
        in_specs=[
            pl.BlockSpec((2, _BLK, 64), lambda i: (0, i, 0)),
            pl.BlockSpec((_BLK, 64), lambda i: (i, 0)),
            pl.BlockSpec((_BLK, 1), lambda i: (i, 0)),
            pl.BlockSpec((1, 64), lambda i: (0, 0)),
        ],
        out_specs=pl.BlockSpec((_BLK, 64), lambda i: (i, 0)),
        out_shape=jax.ShapeDtypeStruct((_N, 64), jnp.float32),
    )(p2, g2p, dinv, b2)


def kernel(x, edge_index, W1, b1, W2, b2):
    ei = edge_index.astype(jnp.int32)
    src, dst = ei[0], ei[1]
    e = src.shape[0]
    ept = -(-e // _NW)
    ch = -(-ept // _K)
    ch = ((ch + 3) // 4) * 4
    pad = _NW * ch * _K - e
    srcp = jnp.concatenate([src, jnp.zeros((pad,), jnp.int32)]).reshape(_NW, ch, _K)
    dstp = jnp.concatenate([dst, jnp.full((pad,), _N, jnp.int32)]).reshape(_NW, ch, _K)
    sdp = jnp.stack([srcp, dstp], axis=2).reshape(_NW * ch, 2, _K)

    degp = _sc_degree(dstp)
    degn = degp[:, :_N, 0:1]
    h1p, dinv = _tc1(x, W1, degn)
    p1 = _sc_scatter(h1p, sdp, 128)[:, :_N, :]
    g2p = _tc2(p1, h1p, dinv, b1.reshape(1, 128), W2)
    p2 = _sc_scatter(g2p, sdp, 64)[:, :_N, :]
    return _tc3(p2, g2p, dinv, b2.reshape(1, 64))


# speedup vs baseline: 1.2498x; 1.2498x over previous
import functools

import jax
import jax.numpy as jnp
from jax import lax
from jax.experimental import pallas as pl
from jax.experimental.pallas import tpu as pltpu
from jax.experimental.pallas import tpu_sc as plsc

_N = 10000
_NPAD = 10240
_K = 128
_NW = 32
_NSUB = 16
_BLK = 1000


def _sc_degree(sdp):
    ch = sdp.shape[0] // _NW
    rpt = _NPAD // _NSUB
    mesh = plsc.VectorSubcoreMesh(core_axis_name="c", subcore_axis_name="s")

    @functools.partial(
        pl.kernel,
        out_type=jax.ShapeDtypeStruct((2, _NPAD, 16), jnp.float32),
        mesh=mesh,
        scratch_types=[
            pltpu.VMEM((2, _K), jnp.int32),
            pltpu.VMEM((2, _K), jnp.int32),
            pltpu.VMEM((_K, 16), jnp.float32),
            pltpu.VMEM((16, 16), jnp.float32),
            pltpu.VMEM_SHARED((_NPAD, 16), jnp.float32),
            pltpu.SemaphoreType.DMA,
            pltpu.SemaphoreType.DMA,
        ],
        compiler_params=pltpu.CompilerParams(use_tc_tiling_on_sc=False),
    )
    def k(sd_h, out_h, s0, s1, ones_v, zero_v, acc, is0, is1):
        cid = lax.axis_index("c")
        sid = lax.axis_index("s")
        wid = cid * _NSUB + sid
        slot = (s0, s1)
        isem = (is0, is1)

        one = jnp.ones((16,), jnp.float32)
        zero = jnp.zeros((16,), jnp.float32)
        for r in range(_K):
            ones_v[r, pl.ds(0, 16)] = one
        for r in range(16):
            zero_v[r, pl.ds(0, 16)] = zero

        for i in range(2):
            pltpu.async_copy(sd_h.at[wid * ch + i], slot[i], isem[i])

        def zbody(z, c):
            pltpu.sync_copy(zero_v, acc.at[pl.ds(sid * rpt + z * 16, 16)])
            return c
        lax.fori_loop(0, rpt // 16, zbody, 0)
        plsc.subcore_barrier()

        def ebody(g, c):
            for u in range(2):
                j = g * 2 + u
                pltpu.make_async_copy(
                    sd_h.at[wid * ch + j], slot[u], isem[u]).wait()
                pltpu.sync_copy(ones_v, acc.at[slot[u].at[1]], add=True)

                @pl.when(j + 2 < ch)
                def _():
                    pltpu.async_copy(sd_h.at[wid * ch + j + 2], slot[u],
                                     isem[u])
            return c
        lax.fori_loop(0, ch // 2, ebody, 0)
        plsc.subcore_barrier()

        pltpu.sync_copy(acc.at[pl.ds(sid * rpt, rpt)],
                        out_h.at[cid, pl.ds(sid * rpt, rpt)])

    return k(sdp)


def _sc_scatter(table_w, sdp, d, kk):
    ch = sdp.shape[0] // _NW
    w = d // 2
    rpt = _NPAD // _NSUB
    zr = 8
    mesh = plsc.VectorSubcoreMesh(core_axis_name="c", subcore_axis_name="s")

    @functools.partial(
        pl.kernel,
        out_type=jax.ShapeDtypeStruct((2, _NPAD, d), jnp.float32),
        mesh=mesh,
        scratch_types=(
            [pltpu.VMEM((2, kk), jnp.int32)] * 8
            + [pltpu.VMEM((kk, w), jnp.int32)] * 2
            + [pltpu.VMEM((kk, d), jnp.float32)] * 2
            + [pltpu.VMEM((zr, d), jnp.float32),
               pltpu.VMEM_SHARED((_NPAD, d), jnp.float32)]
            + [pltpu.SemaphoreType.DMA] * 12
        ),
        compiler_params=pltpu.CompilerParams(use_tc_tiling_on_sc=False,
                                            needs_layout_passes=False),
    )
    def k(table_h, sd_h, out_h,
          i0, i1, i2, i3, i4, i5, i6, i7, rw0, rw1, rf0, rf1, zero_v, acc,
          is0, is1, is2, is3, is4, is5, is6, is7, gs0, gs1, ss0, ss1):
        cid = lax.axis_index("c")
        sid = lax.axis_index("s")
        wid = cid * _NSUB + sid
        slot = (i0, i1, i2, i3, i4, i5, i6, i7)
        isem = (is0, is1, is2, is3, is4, is5, is6, is7)
        rw = (rw0, rw1)
        rf = (rf0, rf1)
        gsem = (gs0, gs1)
        ssem = (ss0, ss1)

        zero = jnp.zeros((16,), jnp.float32)
        for r in range(zr):
            for cc in range(d // 16):
                zero_v[r, pl.ds(cc * 16, 16)] = zero

        for i in range(6):
            pltpu.async_copy(sd_h.at[wid * ch + i], slot[i], isem[i])

        def zbody(z, c):
            pltpu.sync_copy(zero_v, acc.at[pl.ds(sid * rpt + z * zr, zr)])
            return c
        lax.fori_loop(0, rpt // zr, zbody, 0)
        plsc.subcore_barrier()

        for b in range(2):
            pltpu.make_async_copy(sd_h.at[wid * ch + b], slot[b], isem[b]).wait()
            pltpu.async_copy(table_h.at[slot[b].at[0]], rw[b], gsem[b])

        def expand(b):
            def xbody(row, c):
                for cw in range(w // 16):
                    wv = rw[b][row, pl.ds(cw * 16, 16)]
                    lo = plsc.bitcast(jnp.left_shift(wv, 16), jnp.float32)
                    hi = plsc.bitcast(
                        jnp.bitwise_and(wv, jnp.int32(-65536)), jnp.float32)
                    rf[b][row, pl.ds(cw * 32, 16)] = lo
                    rf[b][row, pl.ds(cw * 32 + 16, 16)] = hi
                return c
            lax.fori_loop(0, kk, xbody, 0)

        def ebody(g, c):
            for u in range(8):
                j = g * 8 + u
                b = u % 2
                pltpu.make_async_copy(
                    table_h.at[slot[u].at[0]], rw[b], gsem[b]).wait()

                @pl.when(j + 6 < ch)
                def _():
                    pltpu.async_copy(
                        sd_h.at[wid * ch + j + 6], slot[(u + 6) % 8],
                        isem[(u + 6) % 8])

                expand(b)
                pltpu.sync_copy(rf[b], acc.at[slot[u].at[1]], add=True)

                @pl.when(j + 2 < ch)
                def _():
                    un = (u + 2) % 8
                    pltpu.make_async_copy(
                        sd_h.at[wid * ch + j + 2], slot[un], isem[un]).wait()
                    pltpu.async_copy(table_h.at[slot[un].at[0]], rw[b], gsem[b])
            return c
        lax.fori_loop(0, ch // 8, ebody, 0)
        plsc.subcore_barrier()

        pltpu.sync_copy(acc.at[pl.ds(sid * rpt, rpt)],
                        out_h.at[cid, pl.ds(sid * rpt, rpt)])

    return k(table_w, sdp)


def _pack_rows(h):
    hf = h.astype(jnp.bfloat16).astype(jnp.float32)
    u = lax.bitcast_convert_type(hf, jnp.uint32)
    blocks = []
    for c in range(h.shape[1] // 32):
        lo = lax.shift_right_logical(u[:, 32 * c:32 * c + 16], jnp.uint32(16))
        hi = jnp.bitwise_and(u[:, 32 * c + 16:32 * c + 32],
                             jnp.uint32(0xFFFF0000))
        blocks.append(jnp.bitwise_or(lo, hi))
    return lax.bitcast_convert_type(jnp.concatenate(blocks, axis=1), jnp.int32)


def _tc1(x, w1, degn):
    def body(x_ref, w_ref, dg_ref, h_ref, hw_ref, dv_ref):
        deg = dg_ref[0] + dg_ref[1] + 1.0
        dinv = lax.rsqrt(deg)
        h = jnp.dot(x_ref[...], w_ref[...],
                    preferred_element_type=jnp.float32) * dinv
        h_ref[...] = h
        hw_ref[...] = _pack_rows(h)
        dv_ref[...] = dinv

    return pl.pallas_call(
        body,
        grid=(_N // _BLK,),
        in_specs=[
            pl.BlockSpec((_BLK, 128), lambda i: (i, 0)),
            pl.BlockSpec((128, 128), lambda i: (0, 0)),
            pl.BlockSpec((2, _BLK, 1), lambda i: (0, i, 0)),
        ],
        out_specs=[
            pl.BlockSpec((_BLK, 128), lambda i: (i, 0)),
            pl.BlockSpec((_BLK, 64), lambda i: (i, 0)),
            pl.BlockSpec((_BLK, 1), lambda i: (i, 0)),
        ],
        out_shape=[
            jax.ShapeDtypeStruct((_N, 128), jnp.float32),
            jax.ShapeDtypeStruct((_N, 64), jnp.int32),
            jax.ShapeDtypeStruct((_N, 1), jnp.float32),
        ],
    )(x, w1, degn)


def _tc2(p1, h1p, dinv, b1, w2):
    def body(p_ref, h_ref, dv_ref, b_ref, w_ref, o_ref, ow_ref):
        dinv = dv_ref[...]
        z = dinv * (p_ref[0] + p_ref[1] + h_ref[...]) + b_ref[...]
        h2 = jnp.maximum(z, 0.0)
        g = jnp.dot(h2, w_ref[...],
                    preferred_element_type=jnp.float32) * dinv
        o_ref[...] = g
        ow_ref[...] = _pack_rows(g)

    return pl.pallas_call(
        body,
        grid=(_N // _BLK,),
        in_specs=[
            pl.BlockSpec((2, _BLK, 128), lambda i: (0, i, 0)),
            pl.BlockSpec((_BLK, 128), lambda i: (i, 0)),
            pl.BlockSpec((_BLK, 1), lambda i: (i, 0)),
            pl.BlockSpec((1, 128), lambda i: (0, 0)),
            pl.BlockSpec((128, 64), lambda i: (0, 0)),
        ],
        out_specs=[
            pl.BlockSpec((_BLK, 64), lambda i: (i, 0)),
            pl.BlockSpec((_BLK, 32), lambda i: (i, 0)),
        ],
        out_shape=[
            jax.ShapeDtypeStruct((_N, 64), jnp.float32),
            jax.ShapeDtypeStruct((_N, 32), jnp.int32),
        ],
    )(p1, h1p, dinv, b1, w2)


def _tc3(p2, g2p, dinv, b2):
    def body(p_ref, g_ref, dv_ref, b_ref, o_ref):
        z = dv_ref[...] * (p_ref[0] + p_ref[1] + g_ref[...]) + b_ref[...]
        m = jnp.max(z, axis=1, keepdims=True)
        e = jnp.exp(z - m)
        s = jnp.sum(e, axis=1, keepdims=True)
        o_ref[...] = (z - m) - jnp.log(s)

    return pl.pallas_call(
        body,
        grid=(_N // _BLK,),
        in_specs=[
            pl.BlockSpec((2, _BLK, 64), lambda i: (0, i, 0)),
            pl.BlockSpec((_BLK, 64), lambda i: (i, 0)),
            pl.BlockSpec((_BLK, 1), lambda i: (i, 0)),
            pl.BlockSpec((1, 64), lambda i: (0, 0)),
        ],
        out_specs=pl.BlockSpec((_BLK, 64), lambda i: (i, 0)),
        out_shape=jax.ShapeDtypeStruct((_N, 64), jnp.float32),
    )(p2, g2p, dinv, b2)


def kernel(x, edge_index, W1, b1, W2, b2):
    ei = edge_index.astype(jnp.int32)
    src, dst = ei[0], ei[1]
    e = src.shape[0]
    ept = -(-e // _NW)
    ept = ((ept + 1023) // 1024) * 1024
    pad = _NW * ept - e
    srcp = jnp.concatenate([src, jnp.zeros((pad,), jnp.int32)]).reshape(_NW, ept)
    dstp = jnp.concatenate([dst, jnp.full((pad,), _N, jnp.int32)]).reshape(_NW, ept)
    sd64 = jnp.stack([srcp.reshape(_NW, ept // 64, 64),
                      dstp.reshape(_NW, ept // 64, 64)],
                     axis=2).reshape(_NW * (ept // 64), 2, 64)
    sd128 = jnp.stack([srcp.reshape(_NW, ept // 128, 128),
                       dstp.reshape(_NW, ept // 128, 128)],
                      axis=2).reshape(_NW * (ept // 128), 2, 128)

    degp = _sc_degree(sd128)
    degn = degp[:, :_N, 0:1]
    h1p, h1w, dinv = _tc1(x, W1, degn)
    p1 = _sc_scatter(h1w, sd64, 128, 64)[:, :_N, :]
    g2p, g2w = _tc2(p1, h1p, dinv, b1.reshape(1, 128), W2)
    p2 = _sc_scatter(g2w, sd128, 64, 128)[:, :_N, :]
    return _tc3(p2, g2p, dinv, b2.reshape(1, 64))


# speedup vs baseline: 1.3431x; 1.0746x over previous
import functools

import jax
import jax.numpy as jnp
from jax import lax
from jax.experimental import pallas as pl
from jax.experimental.pallas import tpu as pltpu
from jax.experimental.pallas import tpu_sc as plsc

_N = 10000
_NPAD = 10240
_K = 128
_NW = 32
_NSUB = 16
_BLK = 1000


def _sc_degree(sdp):
    ch = sdp.shape[0] // _NW
    rpt = _NPAD // _NSUB
    mesh = plsc.VectorSubcoreMesh(core_axis_name="c", subcore_axis_name="s")

    @functools.partial(
        pl.kernel,
        out_type=jax.ShapeDtypeStruct((2, _NPAD, 16), jnp.float32),
        mesh=mesh,
        scratch_types=[
            pltpu.VMEM((2, _K), jnp.int32),
            pltpu.VMEM((2, _K), jnp.int32),
            pltpu.VMEM((_K, 16), jnp.float32),
            pltpu.VMEM((16, 16), jnp.float32),
            pltpu.VMEM_SHARED((_NPAD, 16), jnp.float32),
            pltpu.SemaphoreType.DMA,
            pltpu.SemaphoreType.DMA,
        ],
        compiler_params=pltpu.CompilerParams(use_tc_tiling_on_sc=False),
    )
    def k(sd_h, out_h, s0, s1, ones_v, zero_v, acc, is0, is1):
        cid = lax.axis_index("c")
        sid = lax.axis_index("s")
        wid = cid * _NSUB + sid
        slot = (s0, s1)
        isem = (is0, is1)

        one = jnp.ones((16,), jnp.float32)
        zero = jnp.zeros((16,), jnp.float32)
        for r in range(_K):
            ones_v[r, pl.ds(0, 16)] = one
        for r in range(16):
            zero_v[r, pl.ds(0, 16)] = zero

        for i in range(2):
            pltpu.async_copy(sd_h.at[wid * ch + i], slot[i], isem[i])

        def zbody(z, c):
            pltpu.sync_copy(zero_v, acc.at[pl.ds(sid * rpt + z * 16, 16)])
            return c
        lax.fori_loop(0, rpt // 16, zbody, 0)
        plsc.subcore_barrier()

        def ebody(g, c):
            for u in range(2):
                j = g * 2 + u
                pltpu.make_async_copy(
                    sd_h.at[wid * ch + j], slot[u], isem[u]).wait()
                pltpu.sync_copy(ones_v, acc.at[slot[u].at[1]], add=True)

                @pl.when(j + 2 < ch)
                def _():
                    pltpu.async_copy(sd_h.at[wid * ch + j + 2], slot[u],
                                     isem[u])
            return c
        lax.fori_loop(0, ch // 2, ebody, 0)
        plsc.subcore_barrier()

        pltpu.sync_copy(acc.at[pl.ds(sid * rpt, rpt)],
                        out_h.at[cid, pl.ds(sid * rpt, rpt)])

    return k(sdp)


def _sc_scatter(table_w, sdp, d, kk):
    ch = sdp.shape[0] // _NW
    w = d // 2
    rpt = _NPAD // _NSUB
    zr = 8
    mesh = plsc.VectorSubcoreMesh(core_axis_name="c", subcore_axis_name="s")

    @functools.partial(
        pl.kernel,
        out_type=jax.ShapeDtypeStruct((2, _NPAD, d), jnp.float32),
        mesh=mesh,
        scratch_types=(
            [pltpu.VMEM((2, kk), jnp.int32)] * 8
            + [pltpu.VMEM((kk, w), jnp.int32)] * 2
            + [pltpu.VMEM((kk, d), jnp.float32)] * 2
            + [pltpu.VMEM((zr, d), jnp.float32),
               pltpu.VMEM_SHARED((_NPAD, d), jnp.float32)]
            + [pltpu.SemaphoreType.DMA] * 12
        ),
        compiler_params=pltpu.CompilerParams(use_tc_tiling_on_sc=False,
                                            needs_layout_passes=False),
    )
    def k(table_h, sd_h, out_h,
          i0, i1, i2, i3, i4, i5, i6, i7, rw0, rw1, rf0, rf1, zero_v, acc,
          is0, is1, is2, is3, is4, is5, is6, is7, gs0, gs1, ss0, ss1):
        cid = lax.axis_index("c")
        sid = lax.axis_index("s")
        wid = cid * _NSUB + sid
        slot = (i0, i1, i2, i3, i4, i5, i6, i7)
        isem = (is0, is1, is2, is3, is4, is5, is6, is7)
        rw = (rw0, rw1)
        rf = (rf0, rf1)
        gsem = (gs0, gs1)
        ssem = (ss0, ss1)

        zero = jnp.zeros((16,), jnp.float32)
        for r in range(zr):
            for cc in range(d // 16):
                zero_v[r, pl.ds(cc * 16, 16)] = zero

        for i in range(6):
            pltpu.async_copy(sd_h.at[wid * ch + i], slot[i], isem[i])

        def zbody(z, c):
            pltpu.sync_copy(zero_v, acc.at[pl.ds(sid * rpt + z * zr, zr)])
            return c
        lax.fori_loop(0, rpt // zr, zbody, 0)
        plsc.subcore_barrier()

        for b in range(2):
            pltpu.make_async_copy(sd_h.at[wid * ch + b], slot[b], isem[b]).wait()
            pltpu.async_copy(table_h.at[slot[b].at[0]], rw[b], gsem[b])

        def expand(b):
            def xbody(row, c):
                for cw in range(w // 16):
                    wv = rw[b][row, pl.ds(cw * 16, 16)]
                    lo = plsc.bitcast(jnp.left_shift(wv, 16), jnp.float32)
                    hi = plsc.bitcast(
                        jnp.bitwise_and(wv, jnp.int32(-65536)), jnp.float32)
                    rf[b][row, pl.ds(cw * 32, 16)] = lo
                    rf[b][row, pl.ds(cw * 32 + 16, 16)] = hi
                return c
            lax.fori_loop(0, kk, xbody, 0)

        def ebody(g, c):
            for u in range(8):
                j = g * 8 + u
                b = u % 2
                pltpu.make_async_copy(
                    table_h.at[slot[u].at[0]], rw[b], gsem[b]).wait()

                @pl.when(j >= 2)
                def _():
                    pltpu.make_async_copy(
                        rf[b], acc.at[slot[(u + 6) % 8].at[1]], ssem[b]).wait()

                @pl.when(j + 6 < ch)
                def _():
                    pltpu.async_copy(
                        sd_h.at[wid * ch + j + 6], slot[(u + 6) % 8],
                        isem[(u + 6) % 8])

                expand(b)
                pltpu.async_copy(rf[b], acc.at[slot[u].at[1]], ssem[b],
                                 add=True)

                @pl.when(j + 2 < ch)
                def _():
                    un = (u + 2) % 8
                    pltpu.make_async_copy(
                        sd_h.at[wid * ch + j + 2], slot[un], isem[un]).wait()
                    pltpu.async_copy(table_h.at[slot[un].at[0]], rw[b], gsem[b])
            return c
        lax.fori_loop(0, ch // 8, ebody, 0)

        for b in range(2):
            pltpu.make_async_copy(rf[b], acc.at[slot[b].at[1]], ssem[b]).wait()
        plsc.subcore_barrier()

        pltpu.sync_copy(acc.at[pl.ds(sid * rpt, rpt)],
                        out_h.at[cid, pl.ds(sid * rpt, rpt)])

    return k(table_w, sdp)


def _pack_rows(h):
    hf = h.astype(jnp.bfloat16).astype(jnp.float32)
    u = lax.bitcast_convert_type(hf, jnp.uint32)
    blocks = []
    for c in range(h.shape[1] // 32):
        lo = lax.shift_right_logical(u[:, 32 * c:32 * c + 16], jnp.uint32(16))
        hi = jnp.bitwise_and(u[:, 32 * c + 16:32 * c + 32],
                             jnp.uint32(0xFFFF0000))
        blocks.append(jnp.bitwise_or(lo, hi))
    return lax.bitcast_convert_type(jnp.concatenate(blocks, axis=1), jnp.int32)


def _tc1(x, w1, degn):
    def body(x_ref, w_ref, dg_ref, h_ref, hw_ref, dv_ref):
        deg = dg_ref[0] + dg_ref[1] + 1.0
        dinv = lax.rsqrt(deg)
        h = jnp.dot(x_ref[...], w_ref[...],
                    preferred_element_type=jnp.float32) * dinv
        h_ref[...] = h
        hw_ref[...] = _pack_rows(h)
        dv_ref[...] = dinv

    return pl.pallas_call(
        body,
        grid=(_N // _BLK,),
        in_specs=[
            pl.BlockSpec((_BLK, 128), lambda i: (i, 0)),
            pl.BlockSpec((128, 128), lambda i: (0, 0)),
            pl.BlockSpec((2, _BLK, 1), lambda i: (0, i, 0)),
        ],
        out_specs=[
            pl.BlockSpec((_BLK, 128), lambda i: (i, 0)),
            pl.BlockSpec((_BLK, 64), lambda i: (i, 0)),
            pl.BlockSpec((_BLK, 1), lambda i: (i, 0)),
        ],
        out_shape=[
            jax.ShapeDtypeStruct((_N, 128), jnp.float32),
            jax.ShapeDtypeStruct((_N, 64), jnp.int32),
            jax.ShapeDtypeStruct((_N, 1), jnp.float32),
        ],
    )(x, w1, degn)


def _tc2(p1, h1p, dinv, b1, w2):
    def body(p_ref, h_ref, dv_ref, b_ref, w_ref, o_ref, ow_ref):
        dinv = dv_ref[...]
        z = dinv * (p_ref[0] + p_ref[1] + h_ref[...]) + b_ref[...]
        h2 = jnp.maximum(z, 0.0)
        g = jnp.dot(h2, w_ref[...],
                    preferred_element_type=jnp.float32) * dinv
        o_ref[...] = g
        ow_ref[...] = _pack_rows(g)

    return pl.pallas_call(
        body,
        grid=(_N // _BLK,),
        in_specs=[
            pl.BlockSpec((2, _BLK, 128), lambda i: (0, i, 0)),
            pl.BlockSpec((_BLK, 128), lambda i: (i, 0)),
            pl.BlockSpec((_BLK, 1), lambda i: (i, 0)),
            pl.BlockSpec((1, 128), lambda i: (0, 0)),
            pl.BlockSpec((128, 64), lambda i: (0, 0)),
        ],
        out_specs=[
            pl.BlockSpec((_BLK, 64), lambda i: (i, 0)),
            pl.BlockSpec((_BLK, 32), lambda i: (i, 0)),
        ],
        out_shape=[
            jax.ShapeDtypeStruct((_N, 64), jnp.float32),
            jax.ShapeDtypeStruct((_N, 32), jnp.int32),
        ],
    )(p1, h1p, dinv, b1, w2)


def _tc3(p2, g2p, dinv, b2):
    def body(p_ref, g_ref, dv_ref, b_ref, o_ref):
        z = dv_ref[...] * (p_ref[0] + p_ref[1] + g_ref[...]) + b_ref[...]
        m = jnp.max(z, axis=1, keepdims=True)
        e = jnp.exp(z - m)
        s = jnp.sum(e, axis=1, keepdims=True)
        o_ref[...] = (z - m) - jnp.log(s)

    return pl.pallas_call(
        body,
        grid=(_N // _BLK,),
        in_specs=[
            pl.BlockSpec((2, _BLK, 64), lambda i: (0, i, 0)),
            pl.BlockSpec((_BLK, 64), lambda i: (i, 0)),
            pl.BlockSpec((_BLK, 1), lambda i: (i, 0)),
            pl.BlockSpec((1, 64), lambda i: (0, 0)),
        ],
        out_specs=pl.BlockSpec((_BLK, 64), lambda i: (i, 0)),
        out_shape=jax.ShapeDtypeStruct((_N, 64), jnp.float32),
    )(p2, g2p, dinv, b2)


def kernel(x, edge_index, W1, b1, W2, b2):
    ei = edge_index.astype(jnp.int32)
    src, dst = ei[0], ei[1]
    e = src.shape[0]
    ept = -(-e // _NW)
    ept = ((ept + 1023) // 1024) * 1024
    pad = _NW * ept - e
    srcp = jnp.concatenate([src, jnp.zeros((pad,), jnp.int32)]).reshape(_NW, ept)
    dstp = jnp.concatenate([dst, jnp.full((pad,), _N, jnp.int32)]).reshape(_NW, ept)
    sd64 = jnp.stack([srcp.reshape(_NW, ept // 64, 64),
                      dstp.reshape(_NW, ept // 64, 64)],
                     axis=2).reshape(_NW * (ept // 64), 2, 64)
    sd128 = jnp.stack([srcp.reshape(_NW, ept // 128, 128),
                       dstp.reshape(_NW, ept // 128, 128)],
                      axis=2).reshape(_NW * (ept // 128), 2, 128)

    degp = _sc_degree(sd128)
    degn = degp[:, :_N, 0:1]
    h1p, h1w, dinv = _tc1(x, W1, degn)
    p1 = _sc_scatter(h1w, sd64, 128, 64)[:, :_N, :]
    g2p, g2w = _tc2(p1, h1p, dinv, b1.reshape(1, 128), W2)
    p2 = _sc_scatter(g2w, sd128, 64, 128)[:, :_N, :]
    return _tc3(p2, g2p, dinv, b2.reshape(1, 64))
